# Initial kernel scaffold; baseline (speedup 1.0000x reference)
import jax
import jax.numpy as jnp
from jax.experimental import pallas as pl


def kernel(melspecs, centroids):
    raise NotImplementedError("write your pallas kernel here")


# speedup vs baseline: 155.1257x; 155.1257x over previous
import functools

import jax
import jax.numpy as jnp
from jax.experimental import pallas as pl
from jax.experimental.pallas import tpu as pltpu
from jax.experimental.pallas import tpu_sc as plsc

LANES = 16
WIDE = 640
BLK_ROWS = 32


def kernel(melspecs, centroids):
    B, T, M = melspecs.shape
    K = centroids.shape[1]
    R = (B * T * M) // WIDE
    x2 = melspecs.reshape(R, WIDE)
    cent_t = centroids.T

    n_slabs = M // LANES
    reps = WIDE // M

    mesh = plsc.VectorSubcoreMesh(core_axis_name="core", subcore_axis_name="subcore")

    @functools.partial(
        pl.kernel,
        out_type=jax.ShapeDtypeStruct((R, WIDE), jnp.float32),
        mesh=mesh,
        scratch_types=[pltpu.VMEM((K, M), jnp.float32)],
    )
    def run(x_hbm, c_hbm, o_hbm, c_vmem):
        pltpu.sync_copy(c_hbm, c_vmem)

        def body(in_vmem, out_vmem):
            for slab in range(n_slabs):
                cvecs = [c_vmem[k, pl.ds(slab * LANES, LANES)] for k in range(K)]

                @pl.loop(0, BLK_ROWS)
                def _(r, _slab=slab, _cv=cvecs):
                    for j in range(reps):
                        col = (_slab + j * n_slabs) * LANES
                        x = in_vmem[r, pl.ds(col, LANES)]
                        bv = _cv[0]
                        bd = jnp.abs(x - bv)
                        for k in range(1, K):
                            d = jnp.abs(x - _cv[k])
                            m = d < bd
                            bd = jnp.where(m, d, bd)
                            bv = jnp.where(m, _cv[k], bv)
                        out_vmem[r, pl.ds(col, LANES)] = bv

        pltpu.emit_pipeline(
            body,
            grid=(R // BLK_ROWS,),
            in_specs=[pl.BlockSpec((BLK_ROWS, WIDE), lambda i: (i, 0))],
            out_specs=[pl.BlockSpec((BLK_ROWS, WIDE), lambda i: (i, 0))],
            core_axis_name=("core", "subcore"),
            dimension_semantics=(pltpu.PARALLEL,),
        )(x_hbm, o_hbm)

    out = run(x2, cent_t)
    return out.reshape(B, T, M)


# speedup vs baseline: 194.2581x; 1.2523x over previous
import functools

import jax
import jax.numpy as jnp
from jax.experimental import pallas as pl
from jax.experimental.pallas import tpu as pltpu
from jax.experimental.pallas import tpu_sc as plsc

LANES = 16
WIDE = 640
BLK_ROWS = 32

_SORT8 = [
    (0, 1), (2, 3), (4, 5), (6, 7),
    (0, 2), (1, 3), (1, 2), (4, 6), (5, 7), (5, 6),
    (0, 4), (1, 5), (2, 6), (3, 7),
    (2, 4), (3, 5),
    (1, 2), (3, 4), (5, 6),
]


def kernel(melspecs, centroids):
    B, T, M = melspecs.shape
    K = centroids.shape[1]
    R = (B * T * M) // WIDE
    x2 = melspecs.reshape(R, WIDE)
    cent_t = centroids.T

    n_slabs = M // LANES
    reps = WIDE // M

    mesh = plsc.VectorSubcoreMesh(core_axis_name="core", subcore_axis_name="subcore")

    @functools.partial(
        pl.kernel,
        out_type=jax.ShapeDtypeStruct((R, WIDE), jnp.float32),
        mesh=mesh,
        scratch_types=[
            pltpu.VMEM((K, M), jnp.float32),
            pltpu.VMEM((K, M), jnp.float32),
            pltpu.VMEM((K - 1, M), jnp.float32),
        ],
    )
    def run(x_hbm, c_hbm, o_hbm, c_vmem, s_vmem, b_vmem):
        pltpu.sync_copy(c_hbm, c_vmem)
        for slab in range(n_slabs):
            sl = pl.ds(slab * LANES, LANES)
            v = [c_vmem[k, sl] for k in range(K)]
            for i, j in _SORT8:
                lo = jnp.minimum(v[i], v[j])
                hi = jnp.maximum(v[i], v[j])
                v[i], v[j] = lo, hi
            for k in range(K):
                s_vmem[k, sl] = v[k]
            for k in range(K - 1):
                b_vmem[k, sl] = (v[k] + v[k + 1]) * 0.5

        def body(in_vmem, out_vmem):
            for slab in range(n_slabs):
                sl = pl.ds(slab * LANES, LANES)
                sv = [s_vmem[k, sl] for k in range(K)]
                bv = [b_vmem[k, sl] for k in range(K - 1)]

                @pl.loop(0, BLK_ROWS)
                def _(r, _slab=slab, _sv=sv, _bv=bv):
                    for j in range(reps):
                        col = (_slab + j * n_slabs) * LANES
                        x = in_vmem[r, pl.ds(col, LANES)]
                        acc = _sv[0]
                        for k in range(K - 1):
                            acc = jnp.where(x > _bv[k], _sv[k + 1], acc)
                        out_vmem[r, pl.ds(col, LANES)] = acc

        pltpu.emit_pipeline(
            body,
            grid=(R // BLK_ROWS,),
            in_specs=[pl.BlockSpec((BLK_ROWS, WIDE), lambda i: (i, 0))],
            out_specs=[pl.BlockSpec((BLK_ROWS, WIDE), lambda i: (i, 0))],
            core_axis_name=("core", "subcore"),
            dimension_semantics=(pltpu.PARALLEL,),
        )(x_hbm, o_hbm)

    out = run(x2, cent_t)
    return out.reshape(B, T, M)


# speedup vs baseline: 266.8540x; 1.3737x over previous
import functools

import jax
import jax.numpy as jnp
from jax.experimental import pallas as pl
from jax.experimental.pallas import tpu as pltpu
from jax.experimental.pallas import tpu_sc as plsc

LANES = 16
BLK_ROWS = 128

_SORT8 = [
    (0, 1), (2, 3), (4, 5), (6, 7),
    (0, 2), (1, 3), (1, 2), (4, 6), (5, 7), (5, 6),
    (0, 4), (1, 5), (2, 6), (3, 7),
    (2, 4), (3, 5),
    (1, 2), (3, 4), (5, 6),
]


def kernel(melspecs, centroids):
    B, T, M = melspecs.shape
    K = centroids.shape[1]
    R = B * T
    x2 = melspecs.reshape(R, M)
    cent_t = centroids.T

    n_slabs = M // LANES

    mesh = plsc.VectorSubcoreMesh(core_axis_name="core", subcore_axis_name="subcore")

    @functools.partial(
        pl.kernel,
        out_type=jax.ShapeDtypeStruct((R, M), jnp.float32),
        mesh=mesh,
        scratch_types=[
            pltpu.VMEM((K, M), jnp.float32),
            pltpu.VMEM((K, M), jnp.float32),
            pltpu.VMEM((K - 1, M), jnp.float32),
        ],
        compiler_params=pltpu.CompilerParams(use_tc_tiling_on_sc=True),
    )
    def run(x_hbm, c_hbm, o_hbm, c_vmem, s_vmem, b_vmem):
        pltpu.sync_copy(c_hbm, c_vmem)
        for slab in range(n_slabs):
            sl = pl.ds(slab * LANES, LANES)
            v = [c_vmem[k, sl] for k in range(K)]
            for i, j in _SORT8:
                lo = jnp.minimum(v[i], v[j])
                hi = jnp.maximum(v[i], v[j])
                v[i], v[j] = lo, hi
            for k in range(K):
                s_vmem[k, sl] = v[k]
            for k in range(K - 1):
                b_vmem[k, sl] = (v[k] + v[k + 1]) * 0.5

        def body(in_vmem, out_vmem):
            for slab in range(n_slabs):
                sl = pl.ds(slab * LANES, LANES)
                sv = [s_vmem[k, sl] for k in range(K)]
                bv = [b_vmem[k, sl] for k in range(K - 1)]

                @pl.loop(0, BLK_ROWS)
                def _(r, _slab=slab, _sv=sv, _bv=bv):
                    col = _slab * LANES
                    x = in_vmem[r, pl.ds(col, LANES)]
                    acc = _sv[0]
                    for k in range(K - 1):
                        acc = jnp.where(x > _bv[k], _sv[k + 1], acc)
                    out_vmem[r, pl.ds(col, LANES)] = acc

        pltpu.emit_pipeline(
            body,
            grid=(R // BLK_ROWS,),
            in_specs=[pl.BlockSpec((BLK_ROWS, M), lambda i: (i, 0))],
            out_specs=[pl.BlockSpec((BLK_ROWS, M), lambda i: (i, 0))],
            core_axis_name=("core", "subcore"),
            dimension_semantics=(pltpu.PARALLEL,),
        )(x_hbm, o_hbm)

    out = run(x2, cent_t)
    return out.reshape(B, T, M)


# speedup vs baseline: 359.2462x; 1.3462x over previous
import dataclasses
import functools

import jax
import jax.numpy as jnp
from jax import lax
from jax.experimental import pallas as pl
from jax.experimental.pallas import tpu as pltpu
from jax.experimental.pallas import tpu_sc as plsc

LANES = 16
INNER = 4

_SORT8 = [
    (0, 1), (2, 3), (4, 5), (6, 7),
    (0, 2), (1, 3), (1, 2), (4, 6), (5, 7), (5, 6),
    (0, 4), (1, 5), (2, 6), (3, 7),
    (2, 4), (3, 5),
    (1, 2), (3, 4), (5, 6),
]


def kernel(melspecs, centroids):
    B, T, M = melspecs.shape
    K = centroids.shape[1]
    R = B * M
    xt = jnp.transpose(melspecs, (0, 2, 1)).reshape(R, T)
    cent_t = centroids.T

    n_slabs = M // LANES
    n_workers = 32
    rows_per_w = R // n_workers
    chunk_rows = 16
    n_chunks = rows_per_w // chunk_rows

    mesh = plsc.VectorSubcoreMesh(core_axis_name="core", subcore_axis_name="subcore")

    cp = pltpu.CompilerParams()
    if "needs_layout_passes" in pltpu.CompilerParams.__dataclass_fields__:
        cp = dataclasses.replace(cp, needs_layout_passes=False)

    @functools.partial(
        pl.kernel,
        out_type=jax.ShapeDtypeStruct((R, T), jnp.float32),
        mesh=mesh,
        compiler_params=cp,
        scratch_types=[
            pltpu.VMEM((K, M), jnp.float32),
            pltpu.VMEM((K, M), jnp.float32),
            pltpu.VMEM((K - 1, M), jnp.float32),
            pltpu.VMEM((chunk_rows, T), jnp.float32),
            pltpu.VMEM((chunk_rows, T), jnp.float32),
            pltpu.SemaphoreType.DMA,
            pltpu.SemaphoreType.DMA,
            pltpu.SemaphoreType.DMA,
            pltpu.SemaphoreType.DMA,
        ],
    )
    def run(x_hbm, c_hbm, o_hbm, c_vmem, s_vmem, b_vmem,
            buf0, buf1, sin0, sin1, sout0, sout1):
        wid = lax.axis_index("subcore") * 2 + lax.axis_index("core")
        base = wid * rows_per_w
        bufs = (buf0, buf1)
        sins = (sin0, sin1)
        souts = (sout0, sout1)

        pltpu.sync_copy(c_hbm, c_vmem)
        for slab in range(n_slabs):
            sl = pl.ds(slab * LANES, LANES)
            v = [c_vmem[k, sl] for k in range(K)]
            for i, j in _SORT8:
                lo = jnp.minimum(v[i], v[j])
                hi = jnp.maximum(v[i], v[j])
                v[i], v[j] = lo, hi
            for k in range(K):
                s_vmem[k, sl] = v[k]
            for k in range(K - 1):
                b_vmem[k, sl] = (v[k] + v[k + 1]) * 0.5

        def rows_of(c):
            return pl.ds(base + c * chunk_rows, chunk_rows)

        hin = {}
        hout = {}
        hin[0] = pltpu.async_copy(x_hbm.at[rows_of(0)], buf0, sin0)
        for c in range(n_chunks):
            p = c % 2
            if c + 1 < n_chunks:
                q = (c + 1) % 2
                if c + 1 >= 2:
                    hout[q].wait()
                hin[q] = pltpu.async_copy(x_hbm.at[rows_of(c + 1)], bufs[q], sins[q])
            hin[p].wait()
            cur = bufs[p]

            @pl.loop(0, chunk_rows)
            def _(j, _c=c, _cur=cur):
                m = _c * chunk_rows + j
                midx = jnp.full((LANES,), m, jnp.int32)

                def bcast(ref, k):
                    kidx = jnp.full((LANES,), k, jnp.int32)
                    return plsc.load_gather(ref, [kidx, midx])

                sv = [bcast(s_vmem, k) for k in range(K)]
                bv = [bcast(b_vmem, k) for k in range(K - 1)]

                @pl.loop(0, T, step=INNER * LANES)
                def _(col):
                    for u in range(INNER):
                        cs = pl.ds(col + u * LANES, LANES)
                        x = _cur[j, cs]
                        acc = sv[0]
                        for k in range(K - 1):
                            acc = jnp.where(x > bv[k], sv[k + 1], acc)
                        _cur[j, cs] = acc

            hout[p] = pltpu.async_copy(cur, o_hbm.at[rows_of(c)], souts[p])
        hout[0].wait()
        hout[1].wait()

    out = run(xt, cent_t)
    return jnp.transpose(out.reshape(B, M, T), (0, 2, 1))


# speedup vs baseline: 834.0479x; 2.3217x over previous
import dataclasses
import functools

import jax
import jax.numpy as jnp
from jax import lax
from jax.experimental import pallas as pl
from jax.experimental.pallas import tpu as pltpu
from jax.experimental.pallas import tpu_sc as plsc

LANES = 16
INNER = 4
SPLIT = 768
TC_BLK = 256

_SORT8 = [
    (0, 1), (2, 3), (4, 5), (6, 7),
    (0, 2), (1, 3), (1, 2), (4, 6), (5, 7), (5, 6),
    (0, 4), (1, 5), (2, 6), (3, 7),
    (2, 4), (3, 5),
    (1, 2), (3, 4), (5, 6),
]


def _staircase(x, v, b):
    acc = jnp.where(x > b[0], v[1], jnp.broadcast_to(v[0], x.shape))
    for k in range(1, len(b)):
        acc = jnp.where(x > b[k], v[k + 1], acc)
    return acc


def _sort_net(v):
    v = list(v)
    for i, j in _SORT8:
        lo = jnp.minimum(v[i], v[j])
        hi = jnp.maximum(v[i], v[j])
        v[i], v[j] = lo, hi
    return v


def _sc_part(xt, cent_t, T, M, K):
    n_slabs = M // LANES
    n_workers = 32
    rows_per_w = SPLIT // n_workers
    chunk_rows = 8
    n_chunks = rows_per_w // chunk_rows

    mesh = plsc.VectorSubcoreMesh(core_axis_name="core", subcore_axis_name="subcore")

    cp = pltpu.CompilerParams()
    if "needs_layout_passes" in pltpu.CompilerParams.__dataclass_fields__:
        cp = dataclasses.replace(cp, needs_layout_passes=False)

    @functools.partial(
        pl.kernel,
        out_type=jax.ShapeDtypeStruct((SPLIT, T), jnp.float32),
        mesh=mesh,
        compiler_params=cp,
        scratch_types=[
            pltpu.VMEM((K, M), jnp.float32),
            pltpu.VMEM((K, M), jnp.float32),
            pltpu.VMEM((K - 1, M), jnp.float32),
            pltpu.VMEM((chunk_rows, T), jnp.float32),
            pltpu.VMEM((chunk_rows, T), jnp.float32),
            pltpu.VMEM((chunk_rows, T), jnp.float32),
            pltpu.VMEM((chunk_rows, T), jnp.float32),
            pltpu.SemaphoreType.DMA,
            pltpu.SemaphoreType.DMA,
            pltpu.SemaphoreType.DMA,
            pltpu.SemaphoreType.DMA,
        ],
    )
    def run(x_hbm, c_hbm, o_hbm, c_vmem, s_vmem, b_vmem,
            buf0, buf1, obuf0, obuf1, sin0, sin1, sout0, sout1):
        wid = lax.axis_index("subcore") * 2 + lax.axis_index("core")
        base = wid * rows_per_w
        base_m = lax.rem(base, M)
        bufs = (buf0, buf1)
        obufs = (obuf0, obuf1)
        sins = (sin0, sin1)
        souts = (sout0, sout1)

        pltpu.sync_copy(c_hbm, c_vmem)
        for slab in range(n_slabs):
            sl = pl.ds(slab * LANES, LANES)
            v = _sort_net([c_vmem[k, sl] for k in range(K)])
            for k in range(K):
                s_vmem[k, sl] = v[k]
            for k in range(K - 1):
                b_vmem[k, sl] = (v[k] + v[k + 1]) * 0.5

        def rows_of(c):
            return pl.ds(base + c * chunk_rows, chunk_rows)

        hin = {}
        hout = {}
        hin[0] = pltpu.async_copy(x_hbm.at[rows_of(0)], buf0, sin0)
        for c in range(n_chunks):
            p = c % 2
            if c + 1 < n_chunks:
                q = (c + 1) % 2
                if c + 1 >= 2:
                    hout[q].wait()
                hin[q] = pltpu.async_copy(x_hbm.at[rows_of(c + 1)], bufs[q], sins[q])
            hin[p].wait()
            cur = bufs[p]
            ocur = obufs[p]

            @pl.loop(0, chunk_rows)
            def _(j, _c=c, _cur=cur, _ocur=ocur):
                m = lax.rem(base_m + _c * chunk_rows + j, M)
                midx = jnp.full((LANES,), m, jnp.int32)

                def bcast(ref, k):
                    kidx = jnp.full((LANES,), k, jnp.int32)
                    return plsc.load_gather(ref, [kidx, midx])

                sv = [bcast(s_vmem, k) for k in range(K)]
                bv = [bcast(b_vmem, k) for k in range(K - 1)]

                @plsc.parallel_loop(0, T, step=INNER * LANES, unroll=2)
                def _(col):
                    for u in range(INNER):
                        cs = pl.ds(col + u * LANES, LANES)
                        _ocur[j, cs] = _staircase(_cur[j, cs], sv, bv)

            hout[p] = pltpu.async_copy(ocur, o_hbm.at[rows_of(c)], souts[p])
        hout[0].wait()
        hout[1].wait()

    return run(xt, cent_t)


def _tc_part(xt, rawtab, T):
    n_rows = xt.shape[0] - SPLIT
    K = rawtab.shape[1]

    def body(x_ref, tab_ref, o_ref):
        v = _sort_net([tab_ref[:, k:k + 1] for k in range(K)])
        b = [(v[k] + v[k + 1]) * 0.5 for k in range(K - 1)]
        o_ref[...] = _staircase(x_ref[...], v, b)

    return pl.pallas_call(
        body,
        grid=(n_rows // TC_BLK,),
        in_specs=[
            pl.BlockSpec((TC_BLK, T), lambda i: (SPLIT // TC_BLK + i, 0)),
            pl.BlockSpec((TC_BLK, K), lambda i: (i, 0)),
        ],
        out_specs=pl.BlockSpec((TC_BLK, T), lambda i: (i, 0)),
        out_shape=jax.ShapeDtypeStruct((n_rows, T), jnp.float32),
    )(xt, rawtab)


def kernel(melspecs, centroids):
    B, T, M = melspecs.shape
    K = centroids.shape[1]
    R = B * M
    xt = jnp.transpose(melspecs, (0, 2, 1)).reshape(R, T)
    cent_t = centroids.T
    rawtab = jnp.tile(centroids, (B, 1))[SPLIT:]

    sc_out = _sc_part(xt, cent_t, T, M, K)
    tc_out = _tc_part(xt, rawtab, T)
    out = jnp.concatenate([sc_out, tc_out], axis=0)
    return jnp.transpose(out.reshape(B, M, T), (0, 2, 1))
